# Initial kernel scaffold; baseline (speedup 1.0000x reference)
import jax
import jax.numpy as jnp
from jax.experimental import pallas as pl


def kernel(wft_ics, bft_ics, stm, ft_weight, ft_bias, psqt, fc_w, fc_b):
    raise NotImplementedError("write your pallas kernel here")


# speedup vs baseline: 5.2531x; 5.2531x over previous
import jax
import jax.numpy as jnp
from jax import lax
from jax.experimental import pallas as pl
from jax.experimental.pallas import tpu as pltpu
from jax.experimental.pallas import tpu_sc as plsc

NF = 40960
NH = 512
NHV = NH // 16
BATCH = 4096
M = 32
NC, NS = 2, 16
NW = NC * NS
RPW = BATCH // NW


def _body(wft_hbm, bft_hbm, stm_hbm, ftw_hbm, bias_hbm, psqt_hbm, fcw_hbm,
          out_hbm,
          widx, bidx, stmv, biasv, fcwv, outv,
          bufw0, bufb0, psw0, psb0,
          bufw1, bufb1, psw1, psb1,
          sem0, sem1):
    cid = lax.axis_index("c")
    sid = lax.axis_index("s")
    wid = sid * NC + cid
    base = wid * RPW

    pltpu.sync_copy(wft_hbm.at[pl.ds(base, RPW)], widx)
    pltpu.sync_copy(bft_hbm.at[pl.ds(base, RPW)], bidx)
    pltpu.sync_copy(stm_hbm.at[pl.ds(base, RPW)], stmv)
    pltpu.sync_copy(bias_hbm, biasv)
    pltpu.sync_copy(fcw_hbm, fcwv)

    slots = ((bufw0, bufb0, psw0, psb0, sem0),
             (bufw1, bufb1, psw1, psb1, sem1))

    lane0 = lax.iota(jnp.int32, 16) == 0

    def descs(r, slot):
        bw, bb, pw, pb, sem = slots[slot]
        return (
            pltpu.make_async_copy(ftw_hbm.at[widx.at[r]], bw, sem),
            pltpu.make_async_copy(ftw_hbm.at[bidx.at[r]], bb, sem),
            pltpu.make_async_copy(psqt_hbm.at[widx.at[r]], pw, sem),
            pltpu.make_async_copy(psqt_hbm.at[bidx.at[r]], pb, sem),
        )

    def issue(r, slot):
        for d in descs(r, slot):
            d.start()

    def drain(r, slot):
        for d in descs(r, slot):
            d.wait()

    def compute(r, slot):
        bw, bb, pw, pb, _ = slots[slot]
        s = stmv[r][0]

        def col_body(col, carry):
            ca, cb = carry
            o = col * 16

            def j_body(j, vv):
                vw, vb = vv
                return (vw + bw[j, pl.ds(o, 16)], vb + bb[j, pl.ds(o, 16)])

            z = jnp.zeros((16,), jnp.float32)
            vw, vb = lax.fori_loop(0, M, j_body, (z, z))
            b16 = biasv[pl.ds(o, 16)]
            vw = vw + b16
            vb = vb + b16
            x1 = (1.0 - s) * vw + s * vb
            x2 = s * vw + (1.0 - s) * vb
            x1 = jnp.clip(x1, 0.0, 1.0)
            x2 = jnp.clip(x2, 0.0, 1.0)
            ca = ca + (x1 * x1) * fcwv[pl.ds(o, 16)]
            cb = cb + (x2 * x2) * fcwv[pl.ds(NH + o, 16)]
            return ca, cb

        z = jnp.zeros((16,), jnp.float32)
        ca, cb = lax.fori_loop(0, NHV, col_body, (z, z))
        wps = jnp.sum(pw[pl.ds(0, 16)] + pw[pl.ds(16, 16)])
        bps = jnp.sum(pb[pl.ds(0, 16)] + pb[pl.ds(16, 16)])
        res = jnp.sum(ca) + jnp.sum(cb) + (wps - bps) * (0.5 - s)
        idx16 = jnp.full((16,), r, jnp.int32)
        plsc.store_scatter(outv, [idx16],
                           jnp.full((16,), 1.0, jnp.float32) * res, mask=lane0)

    issue(0, 0)

    def row_pair(i, carry):
        r0 = i * 2
        issue(r0 + 1, 1)
        drain(r0, 0)
        compute(r0, 0)

        @pl.when(i < RPW // 2 - 1)
        def _():
            issue(r0 + 2, 0)

        drain(r0 + 1, 1)
        compute(r0 + 1, 1)
        return carry

    lax.fori_loop(0, RPW // 2, row_pair, 0)
    pltpu.sync_copy(outv, out_hbm.at[pl.ds(base, RPW)])


@jax.jit
def _run(wft_ics, bft_ics, stm, ft_weight, ft_bias, psqt, fc_w, fc_b):
    mesh = plsc.VectorSubcoreMesh(core_axis_name="c", subcore_axis_name="s",
                                  num_cores=NC, num_subcores=NS)
    f = pl.kernel(
        _body,
        out_type=jax.ShapeDtypeStruct((BATCH,), jnp.float32),
        mesh=mesh,
        compiler_params=pltpu.CompilerParams(needs_layout_passes=False),
        scratch_types=[
            pltpu.VMEM((RPW, M), jnp.int32),
            pltpu.VMEM((RPW, M), jnp.int32),
            pltpu.VMEM((RPW, 16), jnp.float32),
            pltpu.VMEM((NH,), jnp.float32),
            pltpu.VMEM((2 * NH,), jnp.float32),
            pltpu.VMEM((RPW,), jnp.float32),
            pltpu.VMEM((M, NH), jnp.float32),
            pltpu.VMEM((M, NH), jnp.float32),
            pltpu.VMEM((M,), jnp.float32),
            pltpu.VMEM((M,), jnp.float32),
            pltpu.VMEM((M, NH), jnp.float32),
            pltpu.VMEM((M, NH), jnp.float32),
            pltpu.VMEM((M,), jnp.float32),
            pltpu.VMEM((M,), jnp.float32),
            pltpu.SemaphoreType.DMA,
            pltpu.SemaphoreType.DMA,
        ],
    )
    stm16 = jnp.broadcast_to(stm, (BATCH, 16))
    out = f(wft_ics, bft_ics, stm16, ft_weight, ft_bias, psqt,
            fc_w.reshape(2 * NH))
    return out[:, None] + fc_b


def kernel(wft_ics, bft_ics, stm, ft_weight, ft_bias, psqt, fc_w, fc_b):
    return _run(wft_ics, bft_ics, stm, ft_weight, ft_bias, psqt, fc_w, fc_b)


# speedup vs baseline: 11.1940x; 2.1309x over previous
import jax
import jax.numpy as jnp
from jax import lax
from jax.experimental import pallas as pl
from jax.experimental.pallas import tpu as pltpu
from jax.experimental.pallas import tpu_sc as plsc

NF = 40960
NH = 512
NHV = NH // 16
BATCH = 4096
M = 32
NC, NS = 2, 16
NW = NC * NS
RPW = BATCH // NW


def _body(wft_hbm, bft_hbm, stm_hbm, ftw_hbm, bias_hbm, psqt_hbm, fcw_hbm,
          out_hbm,
          widx, bidx, stmv, biasv, fcwv, outv,
          bufw0, bufb0, psw0, psb0,
          bufw1, bufb1, psw1, psb1,
          sem0, sem1):
    cid = lax.axis_index("c")
    sid = lax.axis_index("s")
    wid = sid * NC + cid
    base = wid * RPW

    pltpu.sync_copy(wft_hbm.at[pl.ds(base, RPW)], widx)
    pltpu.sync_copy(bft_hbm.at[pl.ds(base, RPW)], bidx)
    pltpu.sync_copy(stm_hbm.at[pl.ds(base, RPW)], stmv)
    pltpu.sync_copy(bias_hbm, biasv)
    pltpu.sync_copy(fcw_hbm, fcwv)

    slots = ((bufw0, bufb0, psw0, psb0, sem0),
             (bufw1, bufb1, psw1, psb1, sem1))

    lane0 = lax.iota(jnp.int32, 16) == 0

    def descs(r, slot):
        bw, bb, pw, pb, sem = slots[slot]
        return (
            pltpu.make_async_copy(ftw_hbm.at[widx.at[r]], bw, sem),
            pltpu.make_async_copy(ftw_hbm.at[bidx.at[r]], bb, sem),
            pltpu.make_async_copy(psqt_hbm.at[widx.at[r]], pw, sem),
            pltpu.make_async_copy(psqt_hbm.at[bidx.at[r]], pb, sem),
        )

    def issue(r, slot):
        for d in descs(r, slot):
            d.start()

    def drain(r, slot):
        for d in descs(r, slot):
            d.wait()

    def compute(r, slot):
        bw, bb, pw, pb, _ = slots[slot]
        s = stmv[r][0]

        def col_body(col, carry):
            ca, cb = carry
            o = col * 16

            vw = bw[0, pl.ds(o, 16)]
            vb = bb[0, pl.ds(o, 16)]
            for j in range(1, M):
                vw = vw + bw[j, pl.ds(o, 16)]
                vb = vb + bb[j, pl.ds(o, 16)]
            b16 = biasv[pl.ds(o, 16)]
            vw = vw + b16
            vb = vb + b16
            x1 = (1.0 - s) * vw + s * vb
            x2 = s * vw + (1.0 - s) * vb
            x1 = jnp.clip(x1, 0.0, 1.0)
            x2 = jnp.clip(x2, 0.0, 1.0)
            ca = ca + (x1 * x1) * fcwv[pl.ds(o, 16)]
            cb = cb + (x2 * x2) * fcwv[pl.ds(NH + o, 16)]
            return ca, cb

        z = jnp.zeros((16,), jnp.float32)
        ca, cb = lax.fori_loop(0, NHV, col_body, (z, z))
        wps = jnp.sum(pw[pl.ds(0, 16)] + pw[pl.ds(16, 16)])
        bps = jnp.sum(pb[pl.ds(0, 16)] + pb[pl.ds(16, 16)])
        res = jnp.sum(ca) + jnp.sum(cb) + (wps - bps) * (0.5 - s)
        idx16 = jnp.full((16,), r, jnp.int32)
        plsc.store_scatter(outv, [idx16],
                           jnp.full((16,), 1.0, jnp.float32) * res, mask=lane0)

    issue(0, 0)

    def row_pair(i, carry):
        r0 = i * 2
        issue(r0 + 1, 1)
        drain(r0, 0)
        compute(r0, 0)

        @pl.when(i < RPW // 2 - 1)
        def _():
            issue(r0 + 2, 0)

        drain(r0 + 1, 1)
        compute(r0 + 1, 1)
        return carry

    lax.fori_loop(0, RPW // 2, row_pair, 0)
    pltpu.sync_copy(outv, out_hbm.at[pl.ds(base, RPW)])


@jax.jit
def _run(wft_ics, bft_ics, stm, ft_weight, ft_bias, psqt, fc_w, fc_b):
    mesh = plsc.VectorSubcoreMesh(core_axis_name="c", subcore_axis_name="s",
                                  num_cores=NC, num_subcores=NS)
    f = pl.kernel(
        _body,
        out_type=jax.ShapeDtypeStruct((BATCH,), jnp.float32),
        mesh=mesh,
        compiler_params=pltpu.CompilerParams(needs_layout_passes=False),
        scratch_types=[
            pltpu.VMEM((RPW, M), jnp.int32),
            pltpu.VMEM((RPW, M), jnp.int32),
            pltpu.VMEM((RPW, 16), jnp.float32),
            pltpu.VMEM((NH,), jnp.float32),
            pltpu.VMEM((2 * NH,), jnp.float32),
            pltpu.VMEM((RPW,), jnp.float32),
            pltpu.VMEM((M, NH), jnp.float32),
            pltpu.VMEM((M, NH), jnp.float32),
            pltpu.VMEM((M,), jnp.float32),
            pltpu.VMEM((M,), jnp.float32),
            pltpu.VMEM((M, NH), jnp.float32),
            pltpu.VMEM((M, NH), jnp.float32),
            pltpu.VMEM((M,), jnp.float32),
            pltpu.VMEM((M,), jnp.float32),
            pltpu.SemaphoreType.DMA,
            pltpu.SemaphoreType.DMA,
        ],
    )
    stm16 = jnp.broadcast_to(stm, (BATCH, 16))
    out = f(wft_ics, bft_ics, stm16, ft_weight, ft_bias, psqt,
            fc_w.reshape(2 * NH))
    return out[:, None] + fc_b


def kernel(wft_ics, bft_ics, stm, ft_weight, ft_bias, psqt, fc_w, fc_b):
    return _run(wft_ics, bft_ics, stm, ft_weight, ft_bias, psqt, fc_w, fc_b)


# speedup vs baseline: 11.3173x; 1.0110x over previous
import jax
import jax.numpy as jnp
from jax import lax
from jax.experimental import pallas as pl
from jax.experimental.pallas import tpu as pltpu
from jax.experimental.pallas import tpu_sc as plsc

NF = 40960
NH = 512
NHV = NH // 16
BATCH = 4096
M = 32
NC, NS = 2, 16
NW = NC * NS
RPW = BATCH // NW


def _body(ics_hbm, stm_hbm, ftw_hbm, bias_hbm, psqt_hbm, fcw_hbm,
          out_hbm,
          idxv, stmv, biasv, fcwv, outv,
          buf0, ps0, buf1, ps1, sem0, sem1):
    cid = lax.axis_index("c")
    sid = lax.axis_index("s")
    wid = sid * NC + cid
    base = wid * RPW

    pltpu.sync_copy(ics_hbm.at[pl.ds(base, RPW)], idxv)
    pltpu.sync_copy(stm_hbm.at[pl.ds(base, RPW)], stmv)
    pltpu.sync_copy(bias_hbm, biasv)
    pltpu.sync_copy(fcw_hbm, fcwv)

    slots = ((buf0, ps0, sem0), (buf1, ps1, sem1))
    lane0 = lax.iota(jnp.int32, 16) == 0
    ones = jnp.full((16,), 1.0, jnp.float32)

    def descs(r, slot):
        buf, ps, sem = slots[slot]
        return (pltpu.make_async_copy(ftw_hbm.at[idxv.at[r]], buf, sem),
                pltpu.make_async_copy(psqt_hbm.at[idxv.at[r]], ps, sem))

    def issue(r, slot):
        for d in descs(r, slot):
            d.start()

    def drain(r, slot):
        for d in descs(r, slot):
            d.wait()

    def compute(r, slot):
        buf, ps, _ = slots[slot]
        s = stmv[r][0]

        def col_body(col, carry):
            ca, cb = carry
            o = col * 16
            vw = buf[0, pl.ds(o, 16)]
            vb = buf[M, pl.ds(o, 16)]
            for j in range(1, M):
                vw = vw + buf[j, pl.ds(o, 16)]
                vb = vb + buf[M + j, pl.ds(o, 16)]
            b16 = biasv[pl.ds(o, 16)]
            vw = vw + b16
            vb = vb + b16
            x1 = (1.0 - s) * vw + s * vb
            x2 = s * vw + (1.0 - s) * vb
            x1 = jnp.clip(x1, 0.0, 1.0)
            x2 = jnp.clip(x2, 0.0, 1.0)
            ca = ca + (x1 * x1) * fcwv[pl.ds(o, 16)]
            cb = cb + (x2 * x2) * fcwv[pl.ds(NH + o, 16)]
            return ca, cb

        z = jnp.zeros((16,), jnp.float32)
        ca, cb = lax.fori_loop(0, NHV, col_body, (z, z))
        pg = (ps[pl.ds(0, 16)] + ps[pl.ds(16, 16)]
              - ps[pl.ds(32, 16)] - ps[pl.ds(48, 16)])
        res = jnp.sum(ca) + jnp.sum(cb) + jnp.sum(pg) * (0.5 - s)
        idx16 = jnp.full((16,), r, jnp.int32)
        plsc.store_scatter(outv, [idx16], ones * res, mask=lane0)

    issue(0, 0)

    def row_pair(i, carry):
        r0 = i * 2
        issue(r0 + 1, 1)
        drain(r0, 0)
        compute(r0, 0)

        @pl.when(i < RPW // 2 - 1)
        def _():
            issue(r0 + 2, 0)

        drain(r0 + 1, 1)
        compute(r0 + 1, 1)
        return carry

    lax.fori_loop(0, RPW // 2, row_pair, 0)
    pltpu.sync_copy(outv, out_hbm.at[pl.ds(base, RPW)])


@jax.jit
def _run(wft_ics, bft_ics, stm, ft_weight, ft_bias, psqt, fc_w, fc_b):
    mesh = plsc.VectorSubcoreMesh(core_axis_name="c", subcore_axis_name="s",
                                  num_cores=NC, num_subcores=NS)
    f = pl.kernel(
        _body,
        out_type=jax.ShapeDtypeStruct((BATCH,), jnp.float32),
        mesh=mesh,
        compiler_params=pltpu.CompilerParams(needs_layout_passes=False),
        scratch_types=[
            pltpu.VMEM((RPW, 2 * M), jnp.int32),
            pltpu.VMEM((RPW, 16), jnp.float32),
            pltpu.VMEM((NH,), jnp.float32),
            pltpu.VMEM((2 * NH,), jnp.float32),
            pltpu.VMEM((RPW,), jnp.float32),
            pltpu.VMEM((2 * M, NH), jnp.float32),
            pltpu.VMEM((2 * M,), jnp.float32),
            pltpu.VMEM((2 * M, NH), jnp.float32),
            pltpu.VMEM((2 * M,), jnp.float32),
            pltpu.SemaphoreType.DMA,
            pltpu.SemaphoreType.DMA,
        ],
    )
    ics = jnp.concatenate((wft_ics, bft_ics), axis=1)
    stm16 = jnp.broadcast_to(stm, (BATCH, 16))
    out = f(ics, stm16, ft_weight, ft_bias, psqt, fc_w.reshape(2 * NH))
    return out[:, None] + fc_b


def kernel(wft_ics, bft_ics, stm, ft_weight, ft_bias, psqt, fc_w, fc_b):
    return _run(wft_ics, bft_ics, stm, ft_weight, ft_bias, psqt, fc_w, fc_b)
